# Initial kernel scaffold; baseline (speedup 1.0000x reference)
import jax
import jax.numpy as jnp
from jax.experimental import pallas as pl


def kernel(x, I):
    raise NotImplementedError("write your pallas kernel here")


# speedup vs baseline: 15.5462x; 15.5462x over previous
import jax
import jax.numpy as jnp
from jax import lax
from jax.experimental import pallas as pl
from jax.experimental.pallas import tpu as pltpu
from jax.experimental.pallas import tpu_sc as plsc

C, G, S, L, B = 4, 8192, 8, 4, 16
INFER_STEP = 2
GAMMA = 0.01
INVG = float(1.0 / GAMMA)

NC, NS = 2, 16
NW = NC * NS
NG = 16
RPC = NG * S * L
IDX_ROWS = RPC // 128
TOTAL_ROWS = C * G
ROWS_PER_TILE = TOTAL_ROWS // NW
CHUNKS_PER_TILE = ROWS_PER_TILE // NG


def _sc_body(tab, idx_hbm, m_out, s_out, idx_v, rows_v, mbuf, sbuf, sem):
    wid = lax.axis_index("s") * NC + lax.axis_index("c")
    base_chunk = wid * CHUNKS_PER_TILE

    def chunk_fn(cix, carry):
        q = base_chunk + cix
        pltpu.sync_copy(idx_hbm.at[pl.ds(q * IDX_ROWS, IDX_ROWS)], idx_v)
        cps = [
            pltpu.async_copy(tab.at[idx_v.at[j]], rows_v.at[j], sem)
            for j in range(IDX_ROWS)
        ]
        for cp in cps:
            cp.wait()
        for gl in range(NG):
            r0 = gl * S * L
            ps = []
            for s in range(S):
                k = r0 + s * L
                p = rows_v[k // 128, k % 128]
                for l in range(1, L):
                    p = p * rows_v[(k + l) // 128, (k + l) % 128]
                ps.append(p)
            m = ps[0]
            for s in range(1, S):
                m = jnp.maximum(m, ps[s])
            acc = jnp.exp((ps[0] - m) * INVG)
            for s in range(1, S):
                acc = acc + jnp.exp((ps[s] - m) * INVG)
            mbuf[gl] = m
            sbuf[gl] = acc
        row0 = q * NG
        pltpu.sync_copy(mbuf, m_out.at[pl.ds(row0, NG)])
        pltpu.sync_copy(sbuf, s_out.at[pl.ds(row0, NG)])
        return carry

    lax.fori_loop(0, CHUNKS_PER_TILE, chunk_fn, 0)


_sc_gather = pl.kernel(
    _sc_body,
    out_type=(
        jax.ShapeDtypeStruct((TOTAL_ROWS, B), jnp.float32),
        jax.ShapeDtypeStruct((TOTAL_ROWS, B), jnp.float32),
    ),
    mesh=plsc.VectorSubcoreMesh(
        core_axis_name="c", subcore_axis_name="s", num_cores=NC, num_subcores=NS
    ),
    scratch_types=[
        pltpu.VMEM((IDX_ROWS, 128), jnp.int32),
        pltpu.VMEM((IDX_ROWS, 128, B), jnp.float32),
        pltpu.VMEM((NG, B), jnp.float32),
        pltpu.VMEM((NG, B), jnp.float32),
        pltpu.SemaphoreType.DMA,
    ],
    compiler_params=pltpu.CompilerParams(use_tc_tiling_on_sc=False),
)


def _tc_body(R_ref, m_ref, s_ref, out_ref):
    t = m_ref[:] + GAMMA * jnp.log(s_ref[:])
    mx = jnp.max(t, axis=1, keepdims=True)
    r = t / jnp.maximum(mx, 1.0)
    Rc = R_ref[:]
    mm = jnp.maximum(Rc, r)
    u = mm + GAMMA * jnp.log(
        jnp.exp((Rc - mm) * INVG) + jnp.exp((r - mm) * INVG)
    )
    M = jnp.max(u)
    out_ref[:] = u / jnp.maximum(M, 1.0)


_tc_combine = pl.pallas_call(
    _tc_body,
    out_shape=jax.ShapeDtypeStruct((C, G * B), jnp.float32),
)


def kernel(x, I):
    xT = x.T
    offs = (jnp.arange(C, dtype=jnp.int32) * G)[:, None]
    iadj = (I.reshape(C, G * S * L) + offs).reshape(-1, 128)
    Rt = jnp.broadcast_to(xT.reshape(1, G * B), (C, G * B))
    tab = jnp.tile(xT, (C, 1))
    for _ in range(INFER_STEP):
        m, acc = _sc_gather(tab, iadj)
        Rt = _tc_combine(Rt, m.reshape(C, G * B), acc.reshape(C, G * B))
        tab = Rt.reshape(TOTAL_ROWS, B)
    return Rt.reshape(C, G, B).transpose(0, 2, 1)


# speedup vs baseline: 20.9890x; 1.3501x over previous
import jax
import jax.numpy as jnp
from jax import lax
from jax.experimental import pallas as pl
from jax.experimental.pallas import tpu as pltpu
from jax.experimental.pallas import tpu_sc as plsc

C, G, S, L, B = 4, 8192, 8, 4, 16
INFER_STEP = 2
GAMMA = 0.01
INVG = float(1.0 / GAMMA)

NC, NS = 2, 16
NW = NC * NS
NG = 16
RPC = NG * S * L
IDX_ROWS = RPC // 128
TOTAL_ROWS = C * G
ROWS_PER_TILE = TOTAL_ROWS // NW
NCHUNK = ROWS_PER_TILE // NG
RING = 4


def _sc_body(tab, idx_hbm, m_out, s_out, idx_v, rows_v, mbuf, sbuf, *sems):
    rows_sems = sems[0:4]
    idx_sems = sems[4:8]
    out_sems = sems[8:12]

    wid = lax.axis_index("s") * NC + lax.axis_index("c")
    base = wid * NCHUNK

    def idx_slice(off):
        q = base + jnp.minimum(off, NCHUNK - 1)
        return idx_hbm.at[pl.ds(q * IDX_ROWS, IDX_ROWS)]

    def start_idx(off, p):
        pltpu.async_copy(idx_slice(off), idx_v.at[p], idx_sems[p])

    def wait_idx(p):
        pltpu.make_async_copy(idx_slice(0), idx_v.at[p], idx_sems[p]).wait()

    def start_gathers(p):
        for j in range(IDX_ROWS):
            pltpu.async_copy(tab.at[idx_v.at[p, j]], rows_v.at[p, j],
                             rows_sems[p])

    def wait_gathers(p):
        for j in range(IDX_ROWS):
            pltpu.make_async_copy(tab.at[idx_v.at[p, j]], rows_v.at[p, j],
                                  rows_sems[p]).wait()

    def out_slices(off):
        q = base + off
        return (m_out.at[pl.ds(q * NG, NG)], s_out.at[pl.ds(q * NG, NG)])

    def start_out(off, p):
        mo, so = out_slices(off)
        pltpu.async_copy(mbuf.at[p], mo, out_sems[p])
        pltpu.async_copy(sbuf.at[p], so, out_sems[p])

    def wait_out(p):
        mo, so = out_slices(0)
        pltpu.make_async_copy(mbuf.at[p], mo, out_sems[p]).wait()
        pltpu.make_async_copy(sbuf.at[p], so, out_sems[p]).wait()

    def compute(p):
        for gl in range(NG):
            r0 = gl * S * L
            ps = []
            for s in range(S):
                k = r0 + s * L
                v = rows_v[p, k // 128, k % 128]
                for l in range(1, L):
                    v = v * rows_v[p, (k + l) // 128, (k + l) % 128]
                ps.append(v)
            m = ps[0]
            for s in range(1, S):
                m = jnp.maximum(m, ps[s])
            acc = jnp.exp((ps[0] - m) * INVG)
            for s in range(1, S):
                acc = acc + jnp.exp((ps[s] - m) * INVG)
            mbuf[p, gl] = m
            sbuf[p, gl] = acc

    pltpu.sync_copy(idx_slice(0), idx_v.at[0])
    start_gathers(0)
    start_idx(1, 1)
    wait_idx(1)
    start_gathers(1)
    start_idx(2, 2)

    def outer(i, carry):
        off0 = i * RING
        for u in range(RING):
            off = off0 + u
            p = u
            p2 = (u + 2) % RING
            p3 = (u + 3) % RING
            wait_idx(p2)
            start_gathers(p2)
            wait_gathers(p)
            start_idx(off + 3, p3)
            @pl.when(off >= RING)
            def _():
                wait_out(p)
            compute(p)
            start_out(off, p)
        return carry

    lax.fori_loop(0, NCHUNK // RING, outer, 0)

    wait_gathers(0)
    wait_gathers(1)
    wait_idx(2)
    for p in range(RING):
        wait_out(p)


_sc_gather = pl.kernel(
    _sc_body,
    out_type=(
        jax.ShapeDtypeStruct((TOTAL_ROWS, B), jnp.float32),
        jax.ShapeDtypeStruct((TOTAL_ROWS, B), jnp.float32),
    ),
    mesh=plsc.VectorSubcoreMesh(
        core_axis_name="c", subcore_axis_name="s", num_cores=NC, num_subcores=NS
    ),
    scratch_types=[
        pltpu.VMEM((RING, IDX_ROWS, 128), jnp.int32),
        pltpu.VMEM((RING, IDX_ROWS, 128, B), jnp.float32),
        pltpu.VMEM((RING, NG, B), jnp.float32),
        pltpu.VMEM((RING, NG, B), jnp.float32),
    ] + [pltpu.SemaphoreType.DMA] * 12,
    compiler_params=pltpu.CompilerParams(use_tc_tiling_on_sc=False),
)


def _tc_body(R_ref, m_ref, s_ref, out_ref):
    t = m_ref[:] + GAMMA * jnp.log(s_ref[:])
    mx = jnp.max(t, axis=1, keepdims=True)
    r = t / jnp.maximum(mx, 1.0)
    Rc = R_ref[:]
    mm = jnp.maximum(Rc, r)
    u = mm + GAMMA * jnp.log(
        jnp.exp((Rc - mm) * INVG) + jnp.exp((r - mm) * INVG)
    )
    M = jnp.max(u)
    out_ref[:] = u / jnp.maximum(M, 1.0)


_tc_combine = pl.pallas_call(
    _tc_body,
    out_shape=jax.ShapeDtypeStruct((C, G * B), jnp.float32),
)


def kernel(x, I):
    xT = x.T
    offs = (jnp.arange(C, dtype=jnp.int32) * G)[:, None]
    iadj = (I.reshape(C, G * S * L) + offs).reshape(-1, 128)
    Rt = jnp.broadcast_to(xT.reshape(1, G * B), (C, G * B))
    tab = jnp.tile(xT, (C, 1))
    for _ in range(INFER_STEP):
        m, acc = _sc_gather(tab, iadj)
        Rt = _tc_combine(Rt, m.reshape(C, G * B), acc.reshape(C, G * B))
        tab = Rt.reshape(TOTAL_ROWS, B)
    return Rt.reshape(C, G, B).transpose(0, 2, 1)
